# Initial kernel scaffold; baseline (speedup 1.0000x reference)
import jax
import jax.numpy as jnp
from jax.experimental import pallas as pl


def kernel(node_attrs, coords, edge_index, edge_attrs, W_emb, b_emb, We1, be1, We2, be2, Wa, ba, Wn1, bn1, Wn2, bn2, Wd1, bd1, Wd2, bd2, Wf1, bf1, Wf2, bf2, Wl1, bl1, Wl2, bl2):
    raise NotImplementedError("write your pallas kernel here")


# speedup vs baseline: 3.5549x; 3.5549x over previous
import functools

import jax
import jax.numpy as jnp
from jax import lax
from jax.experimental import pallas as pl
from jax.experimental.pallas import tpu as pltpu
from jax.experimental.pallas import tpu_sc as plsc

_N = 10000
_E = 640000
_HID = 128
_NODE1 = 83
_NC = 2
_NS = 16
_NW = _NC * _NS

_RCHUNK = 2000
_RCH_PER_W = _E // (_NW * _RCHUNK)

_GCHUNK = 512
_NG_CHUNKS = 2 * _E // _GCHUNK
_G_ITERS = (_NG_CHUNKS + _NW - 1) // _NW

_SCHUNK = 256
_SCH_PER_CORE = (_E // 2) // _SCHUNK
_S_ITERS = (_SCH_PER_CORE + _NS - 1) // _NS
_STRIPE = 624
_TAIL = _N - _NS * _STRIPE

_mesh = plsc.VectorSubcoreMesh(
    core_axis_name="c", subcore_axis_name="s", num_cores=_NC, num_subcores=_NS)
_sc_params = pltpu.CompilerParams(needs_layout_passes=False)


@functools.partial(
    pl.kernel,
    out_type=(jax.ShapeDtypeStruct((_E,), jnp.float32),
              jax.ShapeDtypeStruct((2 * _E,), jnp.int32)),
    mesh=_mesh,
    scratch_types=[
        pltpu.VMEM((_N * 3,), jnp.float32),
        pltpu.VMEM((_RCHUNK,), jnp.int32),
        pltpu.VMEM((_RCHUNK,), jnp.int32),
        pltpu.VMEM((_RCHUNK,), jnp.float32),
        pltpu.VMEM((_RCHUNK,), jnp.int32),
    ],
    compiler_params=_sc_params,
)
def _radial_sc(coords_hbm, row_hbm, col_hbm, rad_out, idx2_out,
               cv, rowv, colv, radv, colnv):
    wid = lax.axis_index("s") * _NC + lax.axis_index("c")
    pltpu.sync_copy(coords_hbm, cv)

    def chunk(k, carry):
        base = (wid * _RCH_PER_W + k) * _RCHUNK
        pltpu.sync_copy(row_hbm.at[pl.ds(base, _RCHUNK)], rowv)
        pltpu.sync_copy(col_hbm.at[pl.ds(base, _RCHUNK)], colv)

        def inner(i, c2):
            s = i * 16
            ir = rowv[pl.ds(s, 16)]
            ic = colv[pl.ds(s, 16)]
            fr = ir * 3
            fc = ic * 3
            dx = plsc.load_gather(cv, [fr]) - plsc.load_gather(cv, [fc])
            dy = plsc.load_gather(cv, [fr + 1]) - plsc.load_gather(cv, [fc + 1])
            dz = plsc.load_gather(cv, [fr + 2]) - plsc.load_gather(cv, [fc + 2])
            radv[pl.ds(s, 16)] = dx * dx + dy * dy + dz * dz
            colnv[pl.ds(s, 16)] = ic + _N
            return c2

        lax.fori_loop(0, _RCHUNK // 16, inner, 0)
        pltpu.sync_copy(radv, rad_out.at[pl.ds(base, _RCHUNK)])
        pltpu.sync_copy(rowv, idx2_out.at[pl.ds(base, _RCHUNK)])
        pltpu.sync_copy(colnv, idx2_out.at[pl.ds(_E + base, _RCHUNK)])
        return carry

    lax.fori_loop(0, _RCH_PER_W, chunk, 0)


@functools.partial(
    pl.kernel,
    out_type=jax.ShapeDtypeStruct((2 * _E, _HID), jnp.float32),
    mesh=_mesh,
    scratch_types=[
        pltpu.VMEM((_GCHUNK // 128, 128), jnp.int32),
        pltpu.VMEM((_GCHUNK, _HID), jnp.float32),
        pltpu.SemaphoreType.DMA,
    ],
    compiler_params=_sc_params,
)
def _gather_sc(tab_hbm, idx_hbm, out_hbm, idxv, buf, sem):
    wid = lax.axis_index("s") * _NC + lax.axis_index("c")

    def chunk(k, carry):
        c = wid + k * _NW

        @pl.when(c < _NG_CHUNKS)
        def _():
            pltpu.sync_copy(idx_hbm.at[pl.ds(c * (_GCHUNK // 128), _GCHUNK // 128)],
                            idxv)
            cps = [pltpu.async_copy(tab_hbm.at[idxv.at[j]],
                                    buf.at[pl.ds(j * 128, 128)], sem)
                   for j in range(_GCHUNK // 128)]
            for cp in cps:
                cp.wait()
            pltpu.sync_copy(buf, out_hbm.at[pl.ds(c * _GCHUNK, _GCHUNK)])

        return carry

    lax.fori_loop(0, _G_ITERS, chunk, 0)


@functools.partial(
    pl.kernel,
    out_type=jax.ShapeDtypeStruct((_NC, _N, _HID), jnp.float32),
    mesh=_mesh,
    scratch_types=[
        pltpu.VMEM_SHARED((_N, _HID), jnp.float32),
        pltpu.VMEM((_SCHUNK // 128, 128), jnp.int32),
        pltpu.VMEM((_SCHUNK, _HID), jnp.float32),
    ],
    compiler_params=_sc_params,
)
def _scatter_sc(msg_hbm, row_hbm, zeros_hbm, agg_out, acc, idxv, buf):
    cid = lax.axis_index("c")
    sid = lax.axis_index("s")
    pltpu.sync_copy(zeros_hbm.at[pl.ds(sid * _STRIPE, _STRIPE)],
                    acc.at[pl.ds(sid * _STRIPE, _STRIPE)])

    @pl.when(sid == 0)
    def _():
        pltpu.sync_copy(zeros_hbm.at[pl.ds(_NS * _STRIPE, _TAIL)],
                        acc.at[pl.ds(_NS * _STRIPE, _TAIL)])

    plsc.subcore_barrier()

    def chunk(k, carry):
        local = sid + k * _NS

        @pl.when(local < _SCH_PER_CORE)
        def _():
            c = cid * _SCH_PER_CORE + local
            pltpu.sync_copy(msg_hbm.at[pl.ds(c * _SCHUNK, _SCHUNK)], buf)
            pltpu.sync_copy(row_hbm.at[pl.ds(c * (_SCHUNK // 128), _SCHUNK // 128)],
                            idxv)
            for j in range(_SCHUNK // 128):
                pltpu.sync_copy(buf.at[pl.ds(j * 128, 128)],
                                acc.at[idxv.at[j]], add=True)

        return carry

    lax.fori_loop(0, _S_ITERS, chunk, 0)
    plsc.subcore_barrier()
    pltpu.sync_copy(acc.at[pl.ds(sid * _STRIPE, _STRIPE)],
                    agg_out.at[cid, pl.ds(sid * _STRIPE, _STRIPE)])

    @pl.when(sid == 0)
    def _():
        pltpu.sync_copy(acc.at[pl.ds(_NS * _STRIPE, _TAIL)],
                        agg_out.at[cid, pl.ds(_NS * _STRIPE, _TAIL)])


_BN = 400
_BE = 512


def _emb_body(h0_ref, w_ref, b_ref, o_ref):
    o_ref[...] = jnp.dot(h0_ref[...], w_ref[...],
                         preferred_element_type=jnp.float32) + b_ref[...]


def _emb_tc(h0, W, b):
    return pl.pallas_call(
        _emb_body,
        grid=(_N // _BN,),
        in_specs=[pl.BlockSpec((_BN, _NODE1), lambda i: (i, 0)),
                  pl.BlockSpec((_NODE1, _HID), lambda i: (0, 0)),
                  pl.BlockSpec((1, _HID), lambda i: (0, 0))],
        out_specs=pl.BlockSpec((_BN, _HID), lambda i: (i, 0)),
        out_shape=jax.ShapeDtypeStruct((_N, _HID), jnp.float32),
    )(h0, W, b)


def _ab_body(h_ref, w_ref, o_ref):
    o_ref[...] = jnp.dot(h_ref[...], w_ref[0],
                         preferred_element_type=jnp.float32)


def _ab_tc(h, Wab):
    return pl.pallas_call(
        _ab_body,
        grid=(2, _N // _BN),
        in_specs=[pl.BlockSpec((_BN, _HID), lambda g, j: (j, 0)),
                  pl.BlockSpec((1, _HID, _HID), lambda g, j: (g, 0, 0))],
        out_specs=pl.BlockSpec((_BN, _HID), lambda g, j: (g * (_N // _BN) + j, 0)),
        out_shape=jax.ShapeDtypeStruct((2 * _N, _HID), jnp.float32),
    )(h, Wab)


def _edge_body(g1_ref, g2_ref, r_ref, a_ref, wr_ref, wa_ref, b1_ref,
               w2_ref, b2_ref, watt_ref, batt_ref, o_ref):
    pre = (g1_ref[...] + g2_ref[...]
           + r_ref[...] * wr_ref[...] + a_ref[...] * wa_ref[...] + b1_ref[...])
    m = pre * jax.nn.sigmoid(pre)
    t = jnp.dot(m, w2_ref[...], preferred_element_type=jnp.float32) + b2_ref[...]
    m2 = t * jax.nn.sigmoid(t)
    logit = jnp.sum(m2 * watt_ref[...], axis=1, keepdims=True) + batt_ref[...]
    o_ref[...] = m2 * jax.nn.sigmoid(logit)


def _edge_tc(G, r2, a2, wr, wa, b1, W2, b2, watt, batt):
    nblk = _E // _BE
    return pl.pallas_call(
        _edge_body,
        grid=(nblk,),
        in_specs=[pl.BlockSpec((_BE, _HID), lambda j: (j, 0)),
                  pl.BlockSpec((_BE, _HID), lambda j: (j + nblk, 0)),
                  pl.BlockSpec((_BE, 1), lambda j: (j, 0)),
                  pl.BlockSpec((_BE, 1), lambda j: (j, 0)),
                  pl.BlockSpec((1, _HID), lambda j: (0, 0)),
                  pl.BlockSpec((1, _HID), lambda j: (0, 0)),
                  pl.BlockSpec((1, _HID), lambda j: (0, 0)),
                  pl.BlockSpec((_HID, _HID), lambda j: (0, 0)),
                  pl.BlockSpec((1, _HID), lambda j: (0, 0)),
                  pl.BlockSpec((1, _HID), lambda j: (0, 0)),
                  pl.BlockSpec((1, 1), lambda j: (0, 0))],
        out_specs=pl.BlockSpec((_BE, _HID), lambda j: (j, 0)),
        out_shape=jax.ShapeDtypeStruct((_E, _HID), jnp.float32),
    )(G, G, r2, a2, wr, wa, b1, W2, b2, watt, batt)


def _node_body(h_ref, a0_ref, a1_ref, h0_ref, wh_ref, wg_ref, w0_ref,
               b1_ref, w2_ref, b2_ref, o_ref):
    agg = a0_ref[...] + a1_ref[...]
    t = (jnp.dot(h_ref[...], wh_ref[...], preferred_element_type=jnp.float32)
         + jnp.dot(agg, wg_ref[...], preferred_element_type=jnp.float32)
         + jnp.dot(h0_ref[...], w0_ref[...], preferred_element_type=jnp.float32)
         + b1_ref[...])
    s = t * jax.nn.sigmoid(t)
    o_ref[...] = h_ref[...] + jnp.dot(
        s, w2_ref[...], preferred_element_type=jnp.float32) + b2_ref[...]


def _node_tc(h, a0, a1, h0, Wh, Wg, W0, b1, W2, b2):
    return pl.pallas_call(
        _node_body,
        grid=(_N // _BN,),
        in_specs=[pl.BlockSpec((_BN, _HID), lambda j: (j, 0)),
                  pl.BlockSpec((_BN, _HID), lambda j: (j, 0)),
                  pl.BlockSpec((_BN, _HID), lambda j: (j, 0)),
                  pl.BlockSpec((_BN, _NODE1), lambda j: (j, 0)),
                  pl.BlockSpec((_HID, _HID), lambda j: (0, 0)),
                  pl.BlockSpec((_HID, _HID), lambda j: (0, 0)),
                  pl.BlockSpec((_NODE1, _HID), lambda j: (0, 0)),
                  pl.BlockSpec((1, _HID), lambda j: (0, 0)),
                  pl.BlockSpec((_HID, _HID), lambda j: (0, 0)),
                  pl.BlockSpec((1, _HID), lambda j: (0, 0))],
        out_specs=pl.BlockSpec((_BN, _HID), lambda j: (j, 0)),
        out_shape=jax.ShapeDtypeStruct((_N, _HID), jnp.float32),
    )(h, a0, a1, h0, Wh, Wg, W0, b1, W2, b2)


def _final_body(h_ref, e_ref, wd1_ref, bd1_ref, wd2_ref, bd2_ref,
                wf1_ref, bf1_ref, wf2_ref, bf2_ref,
                wl1a_ref, wl1b_ref, bl1_ref, wl2_ref, bl2_ref, o_ref):
    t = jnp.dot(h_ref[...], wd1_ref[...],
                preferred_element_type=jnp.float32) + bd1_ref[...]
    hd = jnp.dot(t * jax.nn.sigmoid(t), wd2_ref[...],
                 preferred_element_type=jnp.float32) + bd2_ref[...]
    e1 = jax.nn.relu(jnp.dot(e_ref[...], wf1_ref[...],
                             preferred_element_type=jnp.float32) + bf1_ref[...])
    e2 = jax.nn.relu(jnp.dot(e1, wf2_ref[...],
                             preferred_element_type=jnp.float32) + bf2_ref[...])
    u = (jnp.dot(hd, wl1a_ref[...], preferred_element_type=jnp.float32)
         + jnp.dot(e2, wl1b_ref[...], preferred_element_type=jnp.float32)
         + bl1_ref[...])
    u = u * jax.nn.sigmoid(u)
    o_ref[...] = jax.nn.sigmoid(
        jnp.sum(u * wl2_ref[...], axis=1, keepdims=True) + bl2_ref[...])


def _final_tc(h, esm, Wd1, bd1, Wd2, bd2, Wf1, bf1, Wf2, bf2,
              Wl1a, Wl1b, bl1, wl2, bl2):
    i256 = _HID + 128
    return pl.pallas_call(
        _final_body,
        grid=(_N // _BN,),
        in_specs=[pl.BlockSpec((_BN, _HID), lambda j: (j, 0)),
                  pl.BlockSpec((_BN, 1280), lambda j: (j, 0)),
                  pl.BlockSpec((_HID, _HID), lambda j: (0, 0)),
                  pl.BlockSpec((1, _HID), lambda j: (0, 0)),
                  pl.BlockSpec((_HID, _HID), lambda j: (0, 0)),
                  pl.BlockSpec((1, _HID), lambda j: (0, 0)),
                  pl.BlockSpec((1280, 256), lambda j: (0, 0)),
                  pl.BlockSpec((1, 256), lambda j: (0, 0)),
                  pl.BlockSpec((256, 128), lambda j: (0, 0)),
                  pl.BlockSpec((1, 128), lambda j: (0, 0)),
                  pl.BlockSpec((_HID, i256), lambda j: (0, 0)),
                  pl.BlockSpec((128, i256), lambda j: (0, 0)),
                  pl.BlockSpec((1, i256), lambda j: (0, 0)),
                  pl.BlockSpec((1, i256), lambda j: (0, 0)),
                  pl.BlockSpec((1, 1), lambda j: (0, 0))],
        out_specs=pl.BlockSpec((_BN, 1), lambda j: (j, 0)),
        out_shape=jax.ShapeDtypeStruct((_N, 1), jnp.float32),
    )(h, esm, Wd1, bd1, Wd2, bd2, Wf1, bf1, Wf2, bf2,
      Wl1a, Wl1b, bl1, wl2, bl2)


def kernel(node_attrs, coords, edge_index, edge_attrs, W_emb, b_emb,
           We1, be1, We2, be2, Wa, ba, Wn1, bn1, Wn2, bn2, Wd1, bd1,
           Wd2, bd2, Wf1, bf1, Wf2, bf2, Wl1, bl1, Wl2, bl2):
    f32 = jnp.float32
    h0 = node_attrs[:, :_NODE1]
    esm_in = node_attrs[:, _NODE1:]
    row = edge_index[0]
    col = edge_index[1]

    radial, idx2 = _radial_sc(coords.reshape(-1), row, col)
    idx2d = idx2.reshape(2 * _E // 128, 128)
    row2d = row.reshape(_E // 128, 128)
    r2 = radial.reshape(_E, 1)
    a2 = edge_attrs.reshape(_E, 1)
    zeros = jnp.zeros((_N, _HID), f32)

    h = _emb_tc(h0, W_emb, b_emb.reshape(1, _HID))

    for i in range(4):
        Wab = jnp.stack([We1[i, :_HID], We1[i, _HID:2 * _HID]])
        T = _ab_tc(h, Wab)
        G = _gather_sc(T, idx2d)
        msg = _edge_tc(G, r2, a2,
                       We1[i, 2 * _HID].reshape(1, _HID),
                       We1[i, 2 * _HID + 1].reshape(1, _HID),
                       be1[i].reshape(1, _HID),
                       We2[i], be2[i].reshape(1, _HID),
                       Wa[i].reshape(1, _HID), ba[i].reshape(1, 1))
        agg = _scatter_sc(msg, row2d, zeros)
        h = _node_tc(h, agg[0], agg[1], h0,
                     Wn1[i, :_HID], Wn1[i, _HID:2 * _HID], Wn1[i, 2 * _HID:],
                     bn1[i].reshape(1, _HID), Wn2[i], bn2[i].reshape(1, _HID))

    return _final_tc(h, esm_in, Wd1, bd1.reshape(1, _HID), Wd2,
                     bd2.reshape(1, _HID), Wf1, bf1.reshape(1, 256),
                     Wf2, bf2.reshape(1, 128), Wl1[:_HID], Wl1[_HID:],
                     bl1.reshape(1, 256), Wl2.reshape(1, 256),
                     bl2.reshape(1, 1))


# speedup vs baseline: 3.8767x; 1.0905x over previous
import functools

import jax
import jax.numpy as jnp
from jax import lax
from jax.experimental import pallas as pl
from jax.experimental.pallas import tpu as pltpu
from jax.experimental.pallas import tpu_sc as plsc

_N = 10000
_E = 640000
_HID = 128
_NODE1 = 83
_NC = 2
_NS = 16
_NW = _NC * _NS

_RCHUNK = 2000
_RCH_PER_W = _E // (_NW * _RCHUNK)

_GCHUNK = 512
_NG_CHUNKS = 2 * _E // _GCHUNK
_G_ITERS = (_NG_CHUNKS + _NW - 1) // _NW

_SCHUNK = 256
_SCH_PER_CORE = (_E // 2) // _SCHUNK
_S_ITERS = (_SCH_PER_CORE + _NS - 1) // _NS
_STRIPE = 624
_TAIL = _N - _NS * _STRIPE

_mesh = plsc.VectorSubcoreMesh(
    core_axis_name="c", subcore_axis_name="s", num_cores=_NC, num_subcores=_NS)
_sc_params = pltpu.CompilerParams(needs_layout_passes=False)


@functools.partial(
    pl.kernel,
    out_type=(jax.ShapeDtypeStruct((_E,), jnp.float32),
              jax.ShapeDtypeStruct((2 * _E,), jnp.int32)),
    mesh=_mesh,
    scratch_types=[
        pltpu.VMEM((_N * 3,), jnp.float32),
        pltpu.VMEM((_RCHUNK,), jnp.int32),
        pltpu.VMEM((_RCHUNK,), jnp.int32),
        pltpu.VMEM((_RCHUNK,), jnp.float32),
        pltpu.VMEM((_RCHUNK,), jnp.int32),
    ],
    compiler_params=_sc_params,
)
def _radial_sc(coords_hbm, row_hbm, col_hbm, rad_out, idx2_out,
               cv, rowv, colv, radv, colnv):
    wid = lax.axis_index("s") * _NC + lax.axis_index("c")
    pltpu.sync_copy(coords_hbm, cv)

    def chunk(k, carry):
        base = (wid * _RCH_PER_W + k) * _RCHUNK
        pltpu.sync_copy(row_hbm.at[pl.ds(base, _RCHUNK)], rowv)
        pltpu.sync_copy(col_hbm.at[pl.ds(base, _RCHUNK)], colv)

        def inner(i, c2):
            s = i * 16
            ir = rowv[pl.ds(s, 16)]
            ic = colv[pl.ds(s, 16)]
            fr = ir * 3
            fc = ic * 3
            dx = plsc.load_gather(cv, [fr]) - plsc.load_gather(cv, [fc])
            dy = plsc.load_gather(cv, [fr + 1]) - plsc.load_gather(cv, [fc + 1])
            dz = plsc.load_gather(cv, [fr + 2]) - plsc.load_gather(cv, [fc + 2])
            radv[pl.ds(s, 16)] = dx * dx + dy * dy + dz * dz
            colnv[pl.ds(s, 16)] = ic + _N
            return c2

        lax.fori_loop(0, _RCHUNK // 16, inner, 0)
        pltpu.sync_copy(radv, rad_out.at[pl.ds(base, _RCHUNK)])
        pltpu.sync_copy(rowv, idx2_out.at[pl.ds(base, _RCHUNK)])
        pltpu.sync_copy(colnv, idx2_out.at[pl.ds(_E + base, _RCHUNK)])
        return carry

    lax.fori_loop(0, _RCH_PER_W, chunk, 0)


@functools.partial(
    pl.kernel,
    out_type=jax.ShapeDtypeStruct((2 * _E, _HID), jnp.float32),
    mesh=_mesh,
    scratch_types=[
        pltpu.VMEM((_GCHUNK // 128, 128), jnp.int32),
        pltpu.VMEM((_GCHUNK, _HID), jnp.float32),
        pltpu.SemaphoreType.DMA,
    ],
    compiler_params=_sc_params,
)
def _gather_sc(tab_hbm, idx_hbm, out_hbm, idxv, buf, sem):
    wid = lax.axis_index("s") * _NC + lax.axis_index("c")

    def chunk(k, carry):
        c = wid + k * _NW

        @pl.when(c < _NG_CHUNKS)
        def _():
            pltpu.sync_copy(idx_hbm.at[pl.ds(c * (_GCHUNK // 128), _GCHUNK // 128)],
                            idxv)
            cps = [pltpu.async_copy(tab_hbm.at[idxv.at[j]],
                                    buf.at[pl.ds(j * 128, 128)], sem)
                   for j in range(_GCHUNK // 128)]
            for cp in cps:
                cp.wait()
            pltpu.sync_copy(buf, out_hbm.at[pl.ds(c * _GCHUNK, _GCHUNK)])

        return carry

    lax.fori_loop(0, _G_ITERS, chunk, 0)


@functools.partial(
    pl.kernel,
    out_type=jax.ShapeDtypeStruct((_NC, _N, _HID), jnp.float32),
    mesh=_mesh,
    scratch_types=[
        pltpu.VMEM_SHARED((_N, _HID), jnp.float32),
        pltpu.VMEM((_SCHUNK // 128, 128), jnp.int32),
        pltpu.VMEM((_SCHUNK, _HID), jnp.float32),
    ],
    compiler_params=_sc_params,
)
def _scatter_sc(msg_hbm, row_hbm, zeros_hbm, agg_out, acc, idxv, buf):
    cid = lax.axis_index("c")
    sid = lax.axis_index("s")
    pltpu.sync_copy(zeros_hbm.at[pl.ds(sid * _STRIPE, _STRIPE)],
                    acc.at[pl.ds(sid * _STRIPE, _STRIPE)])

    @pl.when(sid == 0)
    def _():
        pltpu.sync_copy(zeros_hbm.at[pl.ds(_NS * _STRIPE, _TAIL)],
                        acc.at[pl.ds(_NS * _STRIPE, _TAIL)])

    plsc.subcore_barrier()

    def chunk(k, carry):
        local = sid + k * _NS

        @pl.when(local < _SCH_PER_CORE)
        def _():
            c = cid * _SCH_PER_CORE + local
            pltpu.sync_copy(msg_hbm.at[pl.ds(c * _SCHUNK, _SCHUNK)], buf)
            pltpu.sync_copy(row_hbm.at[pl.ds(c * (_SCHUNK // 128), _SCHUNK // 128)],
                            idxv)
            for j in range(_SCHUNK // 128):
                pltpu.sync_copy(buf.at[pl.ds(j * 128, 128)],
                                acc.at[idxv.at[j]], add=True)

        return carry

    lax.fori_loop(0, _S_ITERS, chunk, 0)
    plsc.subcore_barrier()
    pltpu.sync_copy(acc.at[pl.ds(sid * _STRIPE, _STRIPE)],
                    agg_out.at[cid, pl.ds(sid * _STRIPE, _STRIPE)])

    @pl.when(sid == 0)
    def _():
        pltpu.sync_copy(acc.at[pl.ds(_NS * _STRIPE, _TAIL)],
                        agg_out.at[cid, pl.ds(_NS * _STRIPE, _TAIL)])


_BN = 400
_BE = 512


def _emb_body(h0_ref, w_ref, b_ref, o_ref):
    o_ref[...] = jnp.dot(h0_ref[...], w_ref[...],
                         preferred_element_type=jnp.float32) + b_ref[...]


def _emb_tc(h0, W, b):
    return pl.pallas_call(
        _emb_body,
        grid=(_N // _BN,),
        in_specs=[pl.BlockSpec((_BN, _NODE1), lambda i: (i, 0)),
                  pl.BlockSpec((_NODE1, _HID), lambda i: (0, 0)),
                  pl.BlockSpec((1, _HID), lambda i: (0, 0))],
        out_specs=pl.BlockSpec((_BN, _HID), lambda i: (i, 0)),
        out_shape=jax.ShapeDtypeStruct((_N, _HID), jnp.float32),
    )(h0, W, b)


def _ab_body(h_ref, w_ref, o_ref):
    o_ref[...] = jnp.dot(h_ref[...], w_ref[0],
                         preferred_element_type=jnp.float32)


def _ab_tc(h, Wab):
    return pl.pallas_call(
        _ab_body,
        grid=(2, _N // _BN),
        in_specs=[pl.BlockSpec((_BN, _HID), lambda g, j: (j, 0)),
                  pl.BlockSpec((1, _HID, _HID), lambda g, j: (g, 0, 0))],
        out_specs=pl.BlockSpec((_BN, _HID), lambda g, j: (g * (_N // _BN) + j, 0)),
        out_shape=jax.ShapeDtypeStruct((2 * _N, _HID), jnp.float32),
    )(h, Wab)


def _sig(x):
    return 0.5 * jnp.tanh(0.5 * x) + 0.5


def _edge_body(g1_ref, g2_ref, r_ref, a_ref, wr_ref, wa_ref, b1_ref,
               w2_ref, b2_ref, watt_ref, batt_ref, o_ref):
    rc = r_ref[...].reshape(_BE, 1)
    ac = a_ref[...].reshape(_BE, 1)
    pre = (g1_ref[...] + g2_ref[...]
           + rc * wr_ref[...] + ac * wa_ref[...] + b1_ref[...])
    m = pre * _sig(pre)
    t = jnp.dot(m, w2_ref[...], preferred_element_type=jnp.float32) + b2_ref[...]
    m2 = t * _sig(t)
    logit = jnp.dot(m2, watt_ref[...],
                    preferred_element_type=jnp.float32) + batt_ref[...]
    o_ref[...] = m2 * _sig(logit)


def _edge_tc(G, r2, a2, wr, wa, b1, W2, b2, watt, batt):
    nblk = _E // _BE
    return pl.pallas_call(
        _edge_body,
        grid=(nblk,),
        in_specs=[pl.BlockSpec((_BE, _HID), lambda j: (j, 0)),
                  pl.BlockSpec((_BE, _HID), lambda j: (j + nblk, 0)),
                  pl.BlockSpec((_BE,), lambda j: (j,)),
                  pl.BlockSpec((_BE,), lambda j: (j,)),
                  pl.BlockSpec((1, _HID), lambda j: (0, 0)),
                  pl.BlockSpec((1, _HID), lambda j: (0, 0)),
                  pl.BlockSpec((1, _HID), lambda j: (0, 0)),
                  pl.BlockSpec((_HID, _HID), lambda j: (0, 0)),
                  pl.BlockSpec((1, _HID), lambda j: (0, 0)),
                  pl.BlockSpec((_HID, 1), lambda j: (0, 0)),
                  pl.BlockSpec((1, 1), lambda j: (0, 0))],
        out_specs=pl.BlockSpec((_BE, _HID), lambda j: (j, 0)),
        out_shape=jax.ShapeDtypeStruct((_E, _HID), jnp.float32),
    )(G, G, r2, a2, wr, wa, b1, W2, b2, watt, batt)


def _node_body(h_ref, a0_ref, a1_ref, h0_ref, wh_ref, wg_ref, w0_ref,
               b1_ref, w2_ref, b2_ref, o_ref):
    agg = a0_ref[...] + a1_ref[...]
    t = (jnp.dot(h_ref[...], wh_ref[...], preferred_element_type=jnp.float32)
         + jnp.dot(agg, wg_ref[...], preferred_element_type=jnp.float32)
         + jnp.dot(h0_ref[...], w0_ref[...], preferred_element_type=jnp.float32)
         + b1_ref[...])
    s = t * jax.nn.sigmoid(t)
    o_ref[...] = h_ref[...] + jnp.dot(
        s, w2_ref[...], preferred_element_type=jnp.float32) + b2_ref[...]


def _node_tc(h, a0, a1, h0, Wh, Wg, W0, b1, W2, b2):
    return pl.pallas_call(
        _node_body,
        grid=(_N // _BN,),
        in_specs=[pl.BlockSpec((_BN, _HID), lambda j: (j, 0)),
                  pl.BlockSpec((_BN, _HID), lambda j: (j, 0)),
                  pl.BlockSpec((_BN, _HID), lambda j: (j, 0)),
                  pl.BlockSpec((_BN, _NODE1), lambda j: (j, 0)),
                  pl.BlockSpec((_HID, _HID), lambda j: (0, 0)),
                  pl.BlockSpec((_HID, _HID), lambda j: (0, 0)),
                  pl.BlockSpec((_NODE1, _HID), lambda j: (0, 0)),
                  pl.BlockSpec((1, _HID), lambda j: (0, 0)),
                  pl.BlockSpec((_HID, _HID), lambda j: (0, 0)),
                  pl.BlockSpec((1, _HID), lambda j: (0, 0))],
        out_specs=pl.BlockSpec((_BN, _HID), lambda j: (j, 0)),
        out_shape=jax.ShapeDtypeStruct((_N, _HID), jnp.float32),
    )(h, a0, a1, h0, Wh, Wg, W0, b1, W2, b2)


def _final_body(h_ref, e_ref, wd1_ref, bd1_ref, wd2_ref, bd2_ref,
                wf1_ref, bf1_ref, wf2_ref, bf2_ref,
                wl1a_ref, wl1b_ref, bl1_ref, wl2_ref, bl2_ref, o_ref):
    t = jnp.dot(h_ref[...], wd1_ref[...],
                preferred_element_type=jnp.float32) + bd1_ref[...]
    hd = jnp.dot(t * jax.nn.sigmoid(t), wd2_ref[...],
                 preferred_element_type=jnp.float32) + bd2_ref[...]
    e1 = jax.nn.relu(jnp.dot(e_ref[...], wf1_ref[...],
                             preferred_element_type=jnp.float32) + bf1_ref[...])
    e2 = jax.nn.relu(jnp.dot(e1, wf2_ref[...],
                             preferred_element_type=jnp.float32) + bf2_ref[...])
    u = (jnp.dot(hd, wl1a_ref[...], preferred_element_type=jnp.float32)
         + jnp.dot(e2, wl1b_ref[...], preferred_element_type=jnp.float32)
         + bl1_ref[...])
    u = u * jax.nn.sigmoid(u)
    o_ref[...] = jax.nn.sigmoid(
        jnp.sum(u * wl2_ref[...], axis=1, keepdims=True) + bl2_ref[...])


def _final_tc(h, esm, Wd1, bd1, Wd2, bd2, Wf1, bf1, Wf2, bf2,
              Wl1a, Wl1b, bl1, wl2, bl2):
    i256 = _HID + 128
    return pl.pallas_call(
        _final_body,
        grid=(_N // _BN,),
        in_specs=[pl.BlockSpec((_BN, _HID), lambda j: (j, 0)),
                  pl.BlockSpec((_BN, 1280), lambda j: (j, 0)),
                  pl.BlockSpec((_HID, _HID), lambda j: (0, 0)),
                  pl.BlockSpec((1, _HID), lambda j: (0, 0)),
                  pl.BlockSpec((_HID, _HID), lambda j: (0, 0)),
                  pl.BlockSpec((1, _HID), lambda j: (0, 0)),
                  pl.BlockSpec((1280, 256), lambda j: (0, 0)),
                  pl.BlockSpec((1, 256), lambda j: (0, 0)),
                  pl.BlockSpec((256, 128), lambda j: (0, 0)),
                  pl.BlockSpec((1, 128), lambda j: (0, 0)),
                  pl.BlockSpec((_HID, i256), lambda j: (0, 0)),
                  pl.BlockSpec((128, i256), lambda j: (0, 0)),
                  pl.BlockSpec((1, i256), lambda j: (0, 0)),
                  pl.BlockSpec((1, i256), lambda j: (0, 0)),
                  pl.BlockSpec((1, 1), lambda j: (0, 0))],
        out_specs=pl.BlockSpec((_BN, 1), lambda j: (j, 0)),
        out_shape=jax.ShapeDtypeStruct((_N, 1), jnp.float32),
    )(h, esm, Wd1, bd1, Wd2, bd2, Wf1, bf1, Wf2, bf2,
      Wl1a, Wl1b, bl1, wl2, bl2)


def kernel(node_attrs, coords, edge_index, edge_attrs, W_emb, b_emb,
           We1, be1, We2, be2, Wa, ba, Wn1, bn1, Wn2, bn2, Wd1, bd1,
           Wd2, bd2, Wf1, bf1, Wf2, bf2, Wl1, bl1, Wl2, bl2):
    f32 = jnp.float32
    h0 = node_attrs[:, :_NODE1]
    esm_in = node_attrs[:, _NODE1:]
    row = edge_index[0]
    col = edge_index[1]

    radial, idx2 = _radial_sc(coords.reshape(-1), row, col)
    idx2d = idx2.reshape(2 * _E // 128, 128)
    row2d = row.reshape(_E // 128, 128)
    zeros = jnp.zeros((_N, _HID), f32)

    h = _emb_tc(h0, W_emb, b_emb.reshape(1, _HID))

    for i in range(4):
        Wab = jnp.stack([We1[i, :_HID], We1[i, _HID:2 * _HID]])
        T = _ab_tc(h, Wab)
        G = _gather_sc(T, idx2d)
        msg = _edge_tc(G, radial, edge_attrs,
                       We1[i, 2 * _HID].reshape(1, _HID),
                       We1[i, 2 * _HID + 1].reshape(1, _HID),
                       be1[i].reshape(1, _HID),
                       We2[i], be2[i].reshape(1, _HID),
                       Wa[i], ba[i].reshape(1, 1))
        agg = _scatter_sc(msg, row2d, zeros)
        h = _node_tc(h, agg[0], agg[1], h0,
                     Wn1[i, :_HID], Wn1[i, _HID:2 * _HID], Wn1[i, 2 * _HID:],
                     bn1[i].reshape(1, _HID), Wn2[i], bn2[i].reshape(1, _HID))

    return _final_tc(h, esm_in, Wd1, bd1.reshape(1, _HID), Wd2,
                     bd2.reshape(1, _HID), Wf1, bf1.reshape(1, 256),
                     Wf2, bf2.reshape(1, 128), Wl1[:_HID], Wl1[_HID:],
                     bl1.reshape(1, 256), Wl2.reshape(1, 256),
                     bl2.reshape(1, 1))
